# baseline (speedup 1.0000x reference)
import functools

import jax
import jax.numpy as jnp
from jax.experimental import pallas as pl
from jax.experimental.pallas import tpu as pltpu

_K = 64
_ROWS = 128

def _body(x_ref, we_ref, be_ref, wd_ref, bd_ref, pre_ref, sp_ref, rec_ref,
          *, k):
    xb = x_ref[...]
    pre = jax.lax.dot_general(
        xb, we_ref[...], (((1,), (1,)), ((), ())),
        preferred_element_type=jnp.float32) + be_ref[...]
    pre_ref[...] = pre

    s = jax.lax.bitcast_convert_type(pre, jnp.int32)
    ks = jnp.where(s >= 0, s, s ^ jnp.int32(0x7FFFFFFF))
    rows, feats = pre.shape
    half = feats // 2

    ks3 = ks.reshape(rows, 2, half)

    def count_ge(arr16, thr):
        m = (arr16 >= thr).astype(jnp.int16)
        m32 = pltpu.bitcast(m, jnp.int32)
        acc = jnp.sum(m32, axis=(1, 2)).reshape(rows, 1, 1)
        return (acc & 0xFFFF) + (acc >> 16)

    hi = (ks3 >> 16).astype(jnp.int16)
    cand = jnp.full((rows, 1, 1), -(2**15), dtype=jnp.int16)
    for bit in range(15, -1, -1):
        inc = jnp.int16(-(2**15) if bit == 15 else 1 << bit)
        t = cand + inc
        cand = jnp.where(count_ge(hi, t) >= k, t, cand)

    m_above = (hi > cand).astype(jnp.int16)
    acc = jnp.sum(pltpu.bitcast(m_above, jnp.int32),
                  axis=(1, 2)).reshape(rows, 1, 1)
    k2 = k - ((acc & 0xFFFF) + (acc >> 16))
    lo = jnp.where(hi == cand, ((ks3 & 0xFFFF) - (2**15)).astype(jnp.int16),
                   jnp.int16(-(2**15)))

    cand2 = jnp.full((rows, 1, 1), -(2**15), dtype=jnp.int16)
    for bit in range(15, -1, -1):
        inc = jnp.int16(-(2**15) if bit == 15 else 1 << bit)
        t = cand2 + inc
        cand2 = jnp.where(count_ge(lo, t) >= k2, t, cand2)

    keep = (hi > cand) | ((hi == cand) & (lo >= cand2))
    sp = jnp.where(keep, pre.reshape(rows, 2, half), 0.0).reshape(rows, feats)
    sp_ref[...] = sp
    rec_ref[...] = jax.lax.dot_general(
        sp, wd_ref[...], (((1,), (1,)), ((), ())),
        preferred_element_type=jnp.float32) + bd_ref[...]


def kernel(x, W_enc, b_enc, W_dec, b_dec):
    n, d = x.shape
    f = W_enc.shape[0]
    r = _ROWS if n % _ROWS == 0 else n
    grid = (n // r,)

    out = pl.pallas_call(
        functools.partial(_body, k=_K),
        grid=grid,
        in_specs=[
            pl.BlockSpec((r, d), lambda i: (i, 0)),
            pl.BlockSpec((f, d), lambda i: (0, 0)),
            pl.BlockSpec((1, f), lambda i: (0, 0)),
            pl.BlockSpec((d, f), lambda i: (0, 0)),
            pl.BlockSpec((1, d), lambda i: (0, 0)),
        ],
        out_specs=[
            pl.BlockSpec((r, f), lambda i: (i, 0)),
            pl.BlockSpec((r, f), lambda i: (i, 0)),
            pl.BlockSpec((r, d), lambda i: (i, 0)),
        ],
        out_shape=[
            jax.ShapeDtypeStruct((n, f), jnp.float32),
            jax.ShapeDtypeStruct((n, f), jnp.float32),
            jax.ShapeDtypeStruct((n, d), jnp.float32),
        ],
        compiler_params=pltpu.CompilerParams(
            dimension_semantics=("arbitrary",),
        ),
    )(x, W_enc, b_enc.reshape(1, f), W_dec, b_dec.reshape(1, d))
    pre_acts, sparse_acts, reconstruction = out
    return (reconstruction, sparse_acts, pre_acts)


# speedup vs baseline: 9.0217x; 9.0217x over previous
import functools

import jax
import jax.numpy as jnp
from jax.experimental import pallas as pl
from jax.experimental.pallas import tpu as pltpu

_K = 64
_ROWS = 256

def _body(x_ref, we_ref, be_ref, wd_ref, bd_ref, pre_ref, sp_ref, rec_ref,
          *, k):
    xb = x_ref[...]
    pre = jax.lax.dot_general(
        xb, we_ref[...], (((1,), (1,)), ((), ())),
        preferred_element_type=jnp.float32) + be_ref[...]
    pre_ref[...] = pre

    s = jax.lax.bitcast_convert_type(pre, jnp.int32)
    ks = jnp.where(s >= 0, s, s ^ jnp.int32(0x7FFFFFFF))

    cand = jnp.full((pre.shape[0], 1), -(2**31), dtype=jnp.int32)
    for bit in range(31, 7, -1):
        inc = jnp.int32(-(2**31) if bit == 31 else 1 << bit)
        t = cand + inc
        cnt = jnp.sum((ks >= t).astype(jnp.int32), axis=1, keepdims=True)
        cand = jnp.where(cnt >= k, t, cand)

    sp = jnp.where(ks >= cand, pre, 0.0)
    sp_ref[...] = sp
    rec_ref[...] = jax.lax.dot_general(
        sp, wd_ref[...], (((1,), (1,)), ((), ())),
        preferred_element_type=jnp.float32) + bd_ref[...]


def kernel(x, W_enc, b_enc, W_dec, b_dec):
    n, d = x.shape
    f = W_enc.shape[0]
    r = _ROWS if n % _ROWS == 0 else n
    grid = (n // r,)

    out = pl.pallas_call(
        functools.partial(_body, k=_K),
        grid=grid,
        in_specs=[
            pl.BlockSpec((r, d), lambda i: (i, 0)),
            pl.BlockSpec((f, d), lambda i: (0, 0)),
            pl.BlockSpec((1, f), lambda i: (0, 0)),
            pl.BlockSpec((d, f), lambda i: (0, 0)),
            pl.BlockSpec((1, d), lambda i: (0, 0)),
        ],
        out_specs=[
            pl.BlockSpec((r, f), lambda i: (i, 0)),
            pl.BlockSpec((r, f), lambda i: (i, 0)),
            pl.BlockSpec((r, d), lambda i: (i, 0)),
        ],
        out_shape=[
            jax.ShapeDtypeStruct((n, f), jnp.float32),
            jax.ShapeDtypeStruct((n, f), jnp.float32),
            jax.ShapeDtypeStruct((n, d), jnp.float32),
        ],
        compiler_params=pltpu.CompilerParams(
            dimension_semantics=("arbitrary",),
        ),
    )(x, W_enc, b_enc.reshape(1, f), W_dec, b_dec.reshape(1, d))
    pre_acts, sparse_acts, reconstruction = out
    return (reconstruction, sparse_acts, pre_acts)


# speedup vs baseline: 9.0648x; 1.0048x over previous
import functools

import jax
import jax.numpy as jnp
from jax.experimental import pallas as pl
from jax.experimental.pallas import tpu as pltpu

_K = 64
_ROWS = 256
_PASSES = 24


def _body(x_ref, we_ref, be_ref, wd_ref, bd_ref, pre_ref, sp_ref, rec_ref,
          buf_ref, *, k):
    i = pl.program_id(0)

    @pl.when(i > 0)
    def _select_and_decode():
        pre = buf_ref[...]
        pre_ref[...] = pre

        s = jax.lax.bitcast_convert_type(pre, jnp.int32)
        ks = jnp.where(s >= 0, s, s ^ jnp.int32(0x7FFFFFFF))

        cand = jnp.full((pre.shape[0], 1), -(2**31), dtype=jnp.int32)
        for bit in range(31, 31 - _PASSES, -1):
            inc = jnp.int32(-(2**31) if bit == 31 else 1 << bit)
            t = cand + inc
            cnt = jnp.sum((ks >= t).astype(jnp.int32), axis=1, keepdims=True)
            cand = jnp.where(cnt >= k, t, cand)

        sp = jnp.where(ks >= cand, pre, 0.0)
        sp_ref[...] = sp
        rec_ref[...] = jax.lax.dot_general(
            sp, wd_ref[...], (((1,), (1,)), ((), ())),
            preferred_element_type=jnp.float32) + bd_ref[...]

    buf_ref[...] = jax.lax.dot_general(
        x_ref[...], we_ref[...], (((1,), (1,)), ((), ())),
        preferred_element_type=jnp.float32) + be_ref[...]


def kernel(x, W_enc, b_enc, W_dec, b_dec):
    n, d = x.shape
    f = W_enc.shape[0]
    r = _ROWS if n % _ROWS == 0 else n
    g = n // r

    out = pl.pallas_call(
        functools.partial(_body, k=_K),
        grid=(g + 1,),
        in_specs=[
            pl.BlockSpec((r, d), lambda i: (jnp.minimum(i, g - 1), 0)),
            pl.BlockSpec((f, d), lambda i: (0, 0)),
            pl.BlockSpec((1, f), lambda i: (0, 0)),
            pl.BlockSpec((d, f), lambda i: (0, 0)),
            pl.BlockSpec((1, d), lambda i: (0, 0)),
        ],
        out_specs=[
            pl.BlockSpec((r, f), lambda i: (jnp.maximum(i - 1, 0), 0)),
            pl.BlockSpec((r, f), lambda i: (jnp.maximum(i - 1, 0), 0)),
            pl.BlockSpec((r, d), lambda i: (jnp.maximum(i - 1, 0), 0)),
        ],
        out_shape=[
            jax.ShapeDtypeStruct((n, f), jnp.float32),
            jax.ShapeDtypeStruct((n, f), jnp.float32),
            jax.ShapeDtypeStruct((n, d), jnp.float32),
        ],
        scratch_shapes=[pltpu.VMEM((r, f), jnp.float32)],
        compiler_params=pltpu.CompilerParams(
            dimension_semantics=("arbitrary",),
        ),
    )(x, W_enc, b_enc.reshape(1, f), W_dec, b_dec.reshape(1, d))
    pre_acts, sparse_acts, reconstruction = out
    return (reconstruction, sparse_acts, pre_acts)


# speedup vs baseline: 11.1590x; 1.2310x over previous
import functools

import jax
import jax.numpy as jnp
from jax.experimental import pallas as pl
from jax.experimental.pallas import tpu as pltpu

_K = 64
_ROWS = 256
_PASSES = 24


def _body(x_ref, we_ref, be_ref, wd_ref, bd_ref, pre_ref, sp_ref, rec_ref,
          buf_ref, *, k):
    i = pl.program_id(0)

    @pl.when(i > 0)
    def _select_and_decode():
        pre = buf_ref[...]
        pre_ref[...] = pre
        rows = pre.shape[0]

        s = jax.lax.bitcast_convert_type(pre, jnp.int32)
        ks = jnp.where(s >= 0, s, s ^ jnp.int32(0x7FFFFFFF))

        def count_ge(arr16, thr32):
            m = jnp.where(arr16 >= thr32.astype(jnp.int16), jnp.int16(1),
                          jnp.int16(0))
            w = m.shape[1]
            while w > 384:
                w //= 2
                m = m[:, :w] + m[:, w:]
            return jnp.sum(m.astype(jnp.int32), axis=1, keepdims=True)

        hi = (ks >> 16).astype(jnp.int16)
        cand = jnp.full((rows, 1), -(2**15), dtype=jnp.int32)
        for bit in range(15, -1, -1):
            t = cand + jnp.int32(1 << bit)
            cand = jnp.where(count_ge(hi, t) >= k, t, cand)

        k2 = k - count_ge(hi, cand + jnp.int32(1))
        lo = jnp.where(hi == cand.astype(jnp.int16),
                       ((ks & 0xFFFF) - (2**15)).astype(jnp.int16),
                       jnp.int16(-(2**15)))
        cand2 = jnp.full((rows, 1), -(2**15), dtype=jnp.int32)
        for bit in range(15, 7, -1):
            t = cand2 + jnp.int32(1 << bit)
            cand2 = jnp.where(count_ge(lo, t) >= k2, t, cand2)

        keep = (hi > cand.astype(jnp.int16)) | (
            (hi == cand.astype(jnp.int16)) & (lo >= cand2.astype(jnp.int16)))
        sp = jnp.where(keep, pre, 0.0)
        sp_ref[...] = sp
        rec_ref[...] = jax.lax.dot_general(
            sp, wd_ref[...], (((1,), (1,)), ((), ())),
            preferred_element_type=jnp.float32) + bd_ref[...]

    buf_ref[...] = jax.lax.dot_general(
        x_ref[...], we_ref[...], (((1,), (1,)), ((), ())),
        preferred_element_type=jnp.float32) + be_ref[...]


def kernel(x, W_enc, b_enc, W_dec, b_dec):
    n, d = x.shape
    f = W_enc.shape[0]
    r = _ROWS if n % _ROWS == 0 else n
    g = n // r

    out = pl.pallas_call(
        functools.partial(_body, k=_K),
        grid=(g + 1,),
        in_specs=[
            pl.BlockSpec((r, d), lambda i: (jnp.minimum(i, g - 1), 0)),
            pl.BlockSpec((f, d), lambda i: (0, 0)),
            pl.BlockSpec((1, f), lambda i: (0, 0)),
            pl.BlockSpec((d, f), lambda i: (0, 0)),
            pl.BlockSpec((1, d), lambda i: (0, 0)),
        ],
        out_specs=[
            pl.BlockSpec((r, f), lambda i: (jnp.maximum(i - 1, 0), 0)),
            pl.BlockSpec((r, f), lambda i: (jnp.maximum(i - 1, 0), 0)),
            pl.BlockSpec((r, d), lambda i: (jnp.maximum(i - 1, 0), 0)),
        ],
        out_shape=[
            jax.ShapeDtypeStruct((n, f), jnp.float32),
            jax.ShapeDtypeStruct((n, f), jnp.float32),
            jax.ShapeDtypeStruct((n, d), jnp.float32),
        ],
        scratch_shapes=[pltpu.VMEM((r, f), jnp.float32)],
        compiler_params=pltpu.CompilerParams(
            dimension_semantics=("arbitrary",),
        ),
    )(x, W_enc, b_enc.reshape(1, f), W_dec, b_dec.reshape(1, d))
    pre_acts, sparse_acts, reconstruction = out
    return (reconstruction, sparse_acts, pre_acts)
